# baseline (speedup 1.0000x reference)
import functools

import jax
import jax.numpy as jnp
from jax.experimental import pallas as pl
from jax.experimental.pallas import tpu as pltpu

_NUM_SELECT = 60
_NBP = 13
_KPAD = 64
_PER_PASS = 4


def _fused_kernel(x_ref, ts_ref, kp_ref, scores_ref, labels_ref, kp26_ref,
                  xs_ref, loc_v, loc_s, kpbuf, sem, rowsem, *, num_classes):
    B, F = x_ref.shape
    NS, D = kp26_ref.shape[1:]
    xs_ref[...] = x_ref[...]
    col = jax.lax.broadcasted_iota(jnp.int32, (B, F), 1)
    lane = jax.lax.broadcasted_iota(jnp.int32, (B, _KPAD), 1)
    lane4 = jax.lax.broadcasted_iota(jnp.int32, (B, _PER_PASS), 1)
    neg = jnp.float32(-jnp.inf)

    def body(i, carry):
        vals, idxs = carry
        x = xs_ref[...]
        loc4 = jnp.zeros((B, _PER_PASS), jnp.int32)
        for j in range(_PER_PASS):
            m = jnp.max(x, axis=1, keepdims=True)
            loc = jnp.min(jnp.where(x >= m, col, F), axis=1, keepdims=True)
            x = jnp.where(col == loc, neg, x)
            sel = lane == i * _PER_PASS + j
            vals = jnp.where(sel, m, vals)
            idxs = jnp.where(sel, loc, idxs)
            loc4 = jnp.where(lane4 == j, loc // num_classes, loc4)
        xs_ref[...] = x
        loc_v[...] = loc4
        cp = pltpu.make_async_copy(loc_v, loc_s, sem)
        cp.start()
        cp.wait()
        for b in range(B):
            for j in range(_PER_PASS):
                pltpu.make_async_copy(
                    kp_ref.at[b, loc_s[b, j]],
                    kpbuf.at[b, i * _PER_PASS + j],
                    rowsem,
                ).start()
        return vals, idxs

    vals = jnp.full((B, _KPAD), neg, jnp.float32)
    idxs = jnp.zeros((B, _KPAD), jnp.int32)
    vals, idxs = jax.lax.fori_loop(
        0, _NUM_SELECT // _PER_PASS, body, (vals, idxs)
    )
    scores_ref[...] = jax.nn.sigmoid(vals)
    labels_ref[...] = idxs % num_classes

    for _ in range(B * NS):
        pltpu.make_async_copy(kp_ref.at[0, 0], kpbuf.at[0, 0], rowsem).wait()
    even = jax.lax.broadcasted_iota(jnp.int32, (NS, D), 1) % 2 == 0
    for b in range(B):
        kp26_ref[b] = kpbuf[b] * jnp.where(even, ts_ref[b, 0], ts_ref[b, 1])


def kernel(pred_logits, pred_keypoints, target_sizes):
    B, N, C = pred_logits.shape
    D = pred_keypoints.shape[-1]
    flat = pred_logits.reshape(B, N * C)

    scores64, labels64, kp26 = pl.pallas_call(
        functools.partial(_fused_kernel, num_classes=C),
        in_specs=[
            pl.BlockSpec(memory_space=pltpu.VMEM),
            pl.BlockSpec(memory_space=pltpu.SMEM),
            pl.BlockSpec(memory_space=pl.ANY),
        ],
        out_shape=[
            jax.ShapeDtypeStruct((B, _KPAD), jnp.float32),
            jax.ShapeDtypeStruct((B, _KPAD), jnp.int32),
            jax.ShapeDtypeStruct((B, _NUM_SELECT, D), jnp.float32),
        ],
        scratch_shapes=[
            pltpu.VMEM((B, N * C), jnp.float32),
            pltpu.VMEM((B, _PER_PASS), jnp.int32),
            pltpu.SMEM((B, _PER_PASS), jnp.int32),
            pltpu.VMEM((B, _NUM_SELECT, D), jnp.float32),
            pltpu.SemaphoreType.DMA,
            pltpu.SemaphoreType.DMA,
        ],
    )(flat, target_sizes, pred_keypoints)

    scores = scores64[:, :_NUM_SELECT]
    labels = labels64[:, :_NUM_SELECT]
    kpr = kp26.reshape(B, _NUM_SELECT, _NBP, 2)
    kpr = jnp.concatenate([kpr, jnp.ones_like(kpr[..., :1])], axis=-1)
    return scores, labels, kpr.reshape(B, _NUM_SELECT, _NBP * 3)
